# baseline (speedup 1.0000x reference)
import functools

import jax
import jax.numpy as jnp
from jax import lax
from jax.experimental import pallas as pl
from jax.experimental.pallas import tpu as pltpu
from jax.experimental.pallas import tpu_sc as plsc

_N = 10000
_E = 320000
_D = 128
_C = 64
_NC, _NS = 2, 16
_NW = _NC * _NS
_EPW = _E // _NW
_CHUNK = 128
_EPW_PAD = 10240
_NCHUNK = _EPW_PAD // _CHUNK
_NA = 10112
_RCHUNK = 40
_NRCHUNK = _N // _RCHUNK
_ZCH = _NA // _CHUNK
_LANES = 16


def _agg_body(with_deg, x_hbm, src_hbm, dst_hbm, *refs):
    if with_deg:
        (out_hbm, deg_hbm, acc_sh, deg_sh, ones_v, src_all, d0, d1,
         rows0, rows1, sem0, sem1, semd0, semd1) = refs
    else:
        (out_hbm, acc_sh, src_all, d0, d1, rows0, rows1,
         sem0, sem1, semd0, semd1) = refs
    zbuf = rows0
    cid = lax.axis_index("c")
    sid = lax.axis_index("s")
    wid = cid * _NS + sid

    zeros16 = jnp.zeros((_LANES,), jnp.float32)
    ones16 = jnp.ones((_LANES,), jnp.float32)

    pltpu.sync_copy(src_hbm.at[wid], src_all)

    def _zrow(r, c):
        for k in range(_D // _LANES):
            zbuf[r, pl.ds(k * _LANES, _LANES)] = zeros16
        return c
    lax.fori_loop(0, _CHUNK, _zrow, 0)
    if with_deg:
        for k in range(_CHUNK // _LANES):
            ones_v[pl.ds(k * _LANES, _LANES)] = ones16
    for j in range((_ZCH + _NS - 1) // _NS):
        rc = sid + _NS * j
        @pl.when(rc < _ZCH)
        def _():
            pltpu.sync_copy(zbuf, acc_sh.at[pl.ds(rc * _CHUNK, _CHUNK), :])
            if with_deg:
                pltpu.sync_copy(zbuf.at[0],
                                deg_sh.at[pl.ds(rc * _CHUNK, _CHUNK)])
    plsc.subcore_barrier()

    def _gather(i, buf, sem):
        pltpu.async_copy(x_hbm.at[src_all.at[pl.ds(i * _CHUNK, _CHUNK)]],
                         buf, sem)

    def _gather_wait(i, buf, sem):
        pltpu.make_async_copy(
            x_hbm.at[src_all.at[pl.ds(i * _CHUNK, _CHUNK)]], buf, sem).wait()

    def _dload(i, buf, sem):
        pltpu.async_copy(dst_hbm.at[wid, i], buf, sem)

    def _dload_wait(i, buf, sem):
        pltpu.make_async_copy(dst_hbm.at[wid, i], buf, sem).wait()

    def _consume(i, buf, dbuf):
        pltpu.sync_copy(buf, acc_sh.at[dbuf], add=True)
        if with_deg:
            pltpu.sync_copy(ones_v, deg_sh.at[dbuf], add=True)

    npair = _NCHUNK // 2
    _dload(0, d0, semd0)
    _dload(1, d1, semd1)
    _gather(0, rows0, sem0)
    _gather(1, rows1, sem1)

    def _pair(j, c):
        i0 = 2 * j
        _gather_wait(i0, rows0, sem0)
        _dload_wait(i0, d0, semd0)
        _consume(i0, rows0, d0)

        @pl.when(j < npair - 1)
        def _():
            _dload(i0 + 2, d0, semd0)
            _gather(i0 + 2, rows0, sem0)
        _gather_wait(i0 + 1, rows1, sem1)
        _dload_wait(i0 + 1, d1, semd1)
        _consume(i0 + 1, rows1, d1)

        @pl.when(j < npair - 1)
        def _():
            _dload(i0 + 3, d1, semd1)
            _gather(i0 + 3, rows1, sem1)
        return c
    lax.fori_loop(0, npair, _pair, 0)
    plsc.subcore_barrier()

    for j in range((_NRCHUNK + _NS - 1) // _NS):
        rc = sid + _NS * j
        @pl.when(rc < _NRCHUNK)
        def _():
            pltpu.sync_copy(
                acc_sh.at[pl.ds(rc * _RCHUNK, _RCHUNK), :],
                out_hbm.at[cid, pl.ds(rc * _RCHUNK, _RCHUNK), :])
    if with_deg:
        @pl.when(sid == 0)
        def _():
            pltpu.sync_copy(deg_sh, deg_hbm.at[cid])


def _make_agg(with_deg):
    mesh = plsc.VectorSubcoreMesh(core_axis_name="c", subcore_axis_name="s")
    out_type = [jax.ShapeDtypeStruct((_NC, _N, _D), jnp.float32)]
    if with_deg:
        out_type.append(jax.ShapeDtypeStruct((_NC, _NA), jnp.float32))
    scratch = [pltpu.VMEM_SHARED((_NA, _D), jnp.float32)]
    if with_deg:
        scratch.append(pltpu.VMEM_SHARED((_NA,), jnp.float32))
        scratch.append(pltpu.VMEM((_CHUNK,), jnp.float32))
    scratch += [
        pltpu.VMEM((_EPW_PAD,), jnp.int32),
        pltpu.VMEM((_CHUNK,), jnp.int32),
        pltpu.VMEM((_CHUNK,), jnp.int32),
        pltpu.VMEM((_CHUNK, _D), jnp.float32),
        pltpu.VMEM((_CHUNK, _D), jnp.float32),
        pltpu.SemaphoreType.DMA,
        pltpu.SemaphoreType.DMA,
        pltpu.SemaphoreType.DMA,
        pltpu.SemaphoreType.DMA,
    ]
    return pl.kernel(
        functools.partial(_agg_body, with_deg),
        out_type=out_type, mesh=mesh, scratch_types=scratch,
        compiler_params=pltpu.CompilerParams(needs_layout_passes=False),
        name="sage_agg_deg" if with_deg else "sage_agg")


_agg_deg = _make_agg(True)
_agg = _make_agg(False)


def _layer1_body(x_ref, p_ref, degp_ref, wr_ref, wn_ref, b_ref, h1_ref):
    deg = jnp.sum(degp_ref[...], axis=1, keepdims=True)
    rdeg = 1.0 / jnp.maximum(deg, 1.0)
    mean = (p_ref[0] + p_ref[1]) * rdeg
    h = (jnp.dot(x_ref[...], wr_ref[...], preferred_element_type=jnp.float32)
         + jnp.dot(mean, wn_ref[...], preferred_element_type=jnp.float32)
         + b_ref[...])
    h1_ref[...] = jnp.maximum(h, 0.0)


def _layer2_body(h1_ref, p_ref, degp_ref, wr_ref, wn_ref, b_ref, wo_ref,
                 bo_ref, out_ref):
    deg = jnp.sum(degp_ref[...], axis=1, keepdims=True)
    rdeg = 1.0 / jnp.maximum(deg, 1.0)
    mean = (p_ref[0] + p_ref[1]) * rdeg
    h1 = h1_ref[...]
    h2 = (jnp.dot(h1, wr_ref[...], preferred_element_type=jnp.float32)
          + jnp.dot(mean, wn_ref[...], preferred_element_type=jnp.float32)
          + b_ref[...])
    h2 = jnp.maximum(h2, 0.0)
    wo = wo_ref[...]
    out_ref[...] = (
        jnp.dot(h1, wo[:_D], preferred_element_type=jnp.float32)
        + jnp.dot(h2, wo[_D:], preferred_element_type=jnp.float32)
        + bo_ref[...])


_R = 2000


def _tc_layer1(x, p, degp_t, W_root1, W_neigh1, b1):
    grid = (_N // _R,)
    return pl.pallas_call(
        _layer1_body,
        grid=grid,
        in_specs=[
            pl.BlockSpec((_R, _D), lambda i: (i, 0)),
            pl.BlockSpec((_NC, _R, _D), lambda i: (0, i, 0)),
            pl.BlockSpec((_R, _NC), lambda i: (i, 0)),
            pl.BlockSpec((_D, _D), lambda i: (0, 0)),
            pl.BlockSpec((_D, _D), lambda i: (0, 0)),
            pl.BlockSpec((1, _D), lambda i: (0, 0)),
        ],
        out_specs=pl.BlockSpec((_R, _D), lambda i: (i, 0)),
        out_shape=jax.ShapeDtypeStruct((_N, _D), jnp.float32),
        name="sage_tc1",
    )(x, p, degp_t, W_root1, W_neigh1, b1.reshape(1, _D))


def _tc_layer2(h1, p, degp_t, W_root2, W_neigh2, b2, W_out, b_out):
    grid = (_N // _R,)
    return pl.pallas_call(
        _layer2_body,
        grid=grid,
        in_specs=[
            pl.BlockSpec((_R, _D), lambda i: (i, 0)),
            pl.BlockSpec((_NC, _R, _D), lambda i: (0, i, 0)),
            pl.BlockSpec((_R, _NC), lambda i: (i, 0)),
            pl.BlockSpec((_D, _D), lambda i: (0, 0)),
            pl.BlockSpec((_D, _D), lambda i: (0, 0)),
            pl.BlockSpec((1, _D), lambda i: (0, 0)),
            pl.BlockSpec((2 * _D, _C), lambda i: (0, 0)),
            pl.BlockSpec((1, _C), lambda i: (0, 0)),
        ],
        out_specs=pl.BlockSpec((_R, _C), lambda i: (i, 0)),
        out_shape=jax.ShapeDtypeStruct((_N, _C), jnp.float32),
        name="sage_tc2",
    )(h1, p, degp_t, W_root2, W_neigh2, b2.reshape(1, _D), W_out,
      b_out.reshape(1, _C))


@jax.jit
def kernel(x, edge_index, W_root1, W_neigh1, b1, W_root2, W_neigh2, b2,
           W_out, b_out):
    pad = _NW * _EPW_PAD - _E
    src = jnp.concatenate(
        [edge_index[0], jnp.zeros((pad,), jnp.int32)]).reshape(_NW, _EPW_PAD)
    dst = jnp.concatenate(
        [edge_index[1], jnp.full((pad,), _N, jnp.int32)]
    ).reshape(_NW, _NCHUNK, _CHUNK)
    p1, degp = _agg_deg(x, src, dst)
    degp_t = degp[:, :_N].T
    h1 = _tc_layer1(x, p1, degp_t, W_root1, W_neigh1, b1)
    (p2,) = _agg(h1, src, dst)
    return _tc_layer2(h1, p2, degp_t, W_root2, W_neigh2, b2, W_out, b_out)


# speedup vs baseline: 4.3410x; 4.3410x over previous
import functools

import jax
import jax.numpy as jnp
from jax import lax
from jax.experimental import pallas as pl
from jax.experimental.pallas import tpu as pltpu
from jax.experimental.pallas import tpu_sc as plsc

_N = 10000
_E = 320000
_D = 128
_C = 64
_NC, _NS = 2, 16
_NW = _NC * _NS
_EPW = _E // _NW
_CHUNK = 64
_NBUF = 4
_NCHUNK = _EPW // _CHUNK
_TAIL = _EPW - _NCHUNK * _CHUNK
_NA = 10112
_RCHUNK = 40
_NRCHUNK = _N // _RCHUNK
_ZCH = _NA // _CHUNK
_LANES = 16


def _agg_body(with_deg, x_hbm, edge_hbm, *refs):
    if with_deg:
        out_hbm, deg_hbm, acc_sh, deg_sh, ones_v, src_all = refs[:6]
        rest = refs[6:]
    else:
        out_hbm, acc_sh, src_all = refs[:3]
        rest = refs[3:]
    dbufs = rest[:_NBUF]
    rowbufs = rest[_NBUF:2 * _NBUF]
    dtail = rest[2 * _NBUF]
    rows_tail = rest[2 * _NBUF + 1]
    gsems = rest[2 * _NBUF + 2:3 * _NBUF + 2]
    dsems = rest[3 * _NBUF + 2:4 * _NBUF + 2]
    zbuf = rowbufs[0]
    cid = lax.axis_index("c")
    sid = lax.axis_index("s")
    wid = cid * _NS + sid

    zeros16 = jnp.zeros((_LANES,), jnp.float32)
    ones16 = jnp.ones((_LANES,), jnp.float32)

    ebase = wid * _EPW
    pltpu.sync_copy(edge_hbm.at[pl.ds(ebase, _EPW)], src_all)

    def _zrow(r, c):
        for k in range(_D // _LANES):
            zbuf[r, pl.ds(k * _LANES, _LANES)] = zeros16
        return c
    lax.fori_loop(0, _CHUNK, _zrow, 0)
    if with_deg:
        for k in range(_CHUNK // _LANES):
            ones_v[pl.ds(k * _LANES, _LANES)] = ones16
    for j in range((_ZCH + _NS - 1) // _NS):
        rc = sid + _NS * j
        @pl.when(rc < _ZCH)
        def _():
            pltpu.sync_copy(zbuf, acc_sh.at[pl.ds(rc * _CHUNK, _CHUNK), :])
    if with_deg:
        for j in range((_NA // _D + _NS - 1) // _NS):
            rc = sid + _NS * j
            @pl.when(rc < _NA // _D)
            def _():
                pltpu.sync_copy(zbuf.at[0], deg_sh.at[pl.ds(rc * _D, _D)])
    plsc.subcore_barrier()

    def _gather(i, buf, sem):
        pltpu.async_copy(x_hbm.at[src_all.at[pl.ds(i * _CHUNK, _CHUNK)]],
                         buf, sem)

    def _gather_wait(i, buf, sem):
        pltpu.make_async_copy(
            x_hbm.at[src_all.at[pl.ds(i * _CHUNK, _CHUNK)]], buf, sem).wait()

    def _dload(i, buf, sem):
        pltpu.async_copy(
            edge_hbm.at[pl.ds(_E + ebase + i * _CHUNK, _CHUNK)], buf, sem)

    def _dload_wait(i, buf, sem):
        pltpu.make_async_copy(
            edge_hbm.at[pl.ds(_E + ebase + i * _CHUNK, _CHUNK)],
            buf, sem).wait()

    def _consume(i, buf, dbuf):
        pltpu.sync_copy(buf, acc_sh.at[dbuf], add=True)
        if with_deg:
            pltpu.sync_copy(ones_v, deg_sh.at[dbuf], add=True)

    ngrp = _NCHUNK // _NBUF
    for b in range(_NBUF):
        _dload(b, dbufs[b], dsems[b])
        _gather(b, rowbufs[b], gsems[b])

    def _grp(j, c):
        i0 = _NBUF * j
        for b in range(_NBUF):
            _gather_wait(i0 + b, rowbufs[b], gsems[b])
            _dload_wait(i0 + b, dbufs[b], dsems[b])
            _consume(i0 + b, rowbufs[b], dbufs[b])

            @pl.when(j < ngrp - 1)
            def _():
                _dload(i0 + b + _NBUF, dbufs[b], dsems[b])
                _gather(i0 + b + _NBUF, rowbufs[b], gsems[b])
        return c
    lax.fori_loop(0, ngrp, _grp, 0)

    tbase = ebase + _NCHUNK * _CHUNK
    pltpu.sync_copy(edge_hbm.at[pl.ds(_E + tbase, _TAIL)], dtail)
    pltpu.sync_copy(
        x_hbm.at[src_all.at[pl.ds(_NCHUNK * _CHUNK, _TAIL)]], rows_tail)
    pltpu.sync_copy(rows_tail, acc_sh.at[dtail], add=True)
    if with_deg:
        pltpu.sync_copy(ones_v.at[pl.ds(0, _TAIL)], deg_sh.at[dtail],
                        add=True)
    plsc.subcore_barrier()

    for j in range((_NRCHUNK + _NS - 1) // _NS):
        rc = sid + _NS * j
        @pl.when(rc < _NRCHUNK)
        def _():
            pltpu.sync_copy(
                acc_sh.at[pl.ds(rc * _RCHUNK, _RCHUNK), :],
                out_hbm.at[cid, pl.ds(rc * _RCHUNK, _RCHUNK), :])
    if with_deg:
        @pl.when(sid == 0)
        def _():
            pltpu.sync_copy(deg_sh, deg_hbm.at[cid])


def _make_agg(with_deg):
    mesh = plsc.VectorSubcoreMesh(core_axis_name="c", subcore_axis_name="s")
    out_type = [jax.ShapeDtypeStruct((_NC, _N, _D), jnp.float32)]
    if with_deg:
        out_type.append(jax.ShapeDtypeStruct((_NC, _NA), jnp.float32))
    scratch = [pltpu.VMEM_SHARED((_NA, _D), jnp.float32)]
    if with_deg:
        scratch.append(pltpu.VMEM_SHARED((_NA,), jnp.float32))
        scratch.append(pltpu.VMEM((_CHUNK,), jnp.float32))
    scratch.append(pltpu.VMEM((_EPW,), jnp.int32))
    scratch += [pltpu.VMEM((_CHUNK,), jnp.int32) for _ in range(_NBUF)]
    scratch += [pltpu.VMEM((_CHUNK, _D), jnp.float32) for _ in range(_NBUF)]
    scratch.append(pltpu.VMEM((_TAIL,), jnp.int32))
    scratch.append(pltpu.VMEM((_TAIL, _D), jnp.float32))
    scratch += [pltpu.SemaphoreType.DMA for _ in range(2 * _NBUF)]
    return pl.kernel(
        functools.partial(_agg_body, with_deg),
        out_type=out_type, mesh=mesh, scratch_types=scratch,
        compiler_params=pltpu.CompilerParams(needs_layout_passes=False),
        name="sage_agg_deg" if with_deg else "sage_agg")


_agg_deg = _make_agg(True)
_agg = _make_agg(False)


def _root1_body(x_ref, wr_ref, b_ref, r_ref):
    r_ref[...] = (
        jnp.dot(x_ref[...], wr_ref[...], preferred_element_type=jnp.float32)
        + b_ref[...])


def _layer1_body(r_ref, p_ref, degp_ref, wn_ref, h1_ref):
    deg = jnp.sum(degp_ref[...], axis=1, keepdims=True)
    rdeg = 1.0 / jnp.maximum(deg, 1.0)
    mean = (p_ref[0] + p_ref[1]) * rdeg
    h = r_ref[...] + jnp.dot(mean, wn_ref[...],
                             preferred_element_type=jnp.float32)
    h1_ref[...] = jnp.maximum(h, 0.0)


def _root2_body(h1_ref, wr_ref, b_ref, wo_ref, bo_ref, r_ref, o_ref):
    h1 = h1_ref[...]
    r_ref[...] = (
        jnp.dot(h1, wr_ref[...], preferred_element_type=jnp.float32)
        + b_ref[...])
    o_ref[...] = (
        jnp.dot(h1, wo_ref[...][:_D], preferred_element_type=jnp.float32)
        + bo_ref[...])


def _layer2_body(r_ref, o_ref, p_ref, degp_ref, wn_ref, wo_ref, out_ref):
    deg = jnp.sum(degp_ref[...], axis=1, keepdims=True)
    rdeg = 1.0 / jnp.maximum(deg, 1.0)
    mean = (p_ref[0] + p_ref[1]) * rdeg
    h2 = r_ref[...] + jnp.dot(mean, wn_ref[...],
                              preferred_element_type=jnp.float32)
    h2 = jnp.maximum(h2, 0.0)
    out_ref[...] = o_ref[...] + jnp.dot(
        h2, wo_ref[...][_D:], preferred_element_type=jnp.float32)


_R = 2000

_row_spec = pl.BlockSpec((_R, _D), lambda i: (i, 0))
_p_spec = pl.BlockSpec((_NC, _R, _D), lambda i: (0, i, 0))
_degp_spec = pl.BlockSpec((_R, _NC), lambda i: (i, 0))
_w_spec = pl.BlockSpec((_D, _D), lambda i: (0, 0))
_b_spec = pl.BlockSpec((1, _D), lambda i: (0, 0))
_wo_spec = pl.BlockSpec((2 * _D, _C), lambda i: (0, 0))
_bo_spec = pl.BlockSpec((1, _C), lambda i: (0, 0))
_out_spec = pl.BlockSpec((_R, _C), lambda i: (i, 0))


def _tc_root1(x, W_root1, b1):
    return pl.pallas_call(
        _root1_body, grid=(_N // _R,),
        in_specs=[_row_spec, _w_spec, _b_spec],
        out_specs=_row_spec,
        out_shape=jax.ShapeDtypeStruct((_N, _D), jnp.float32),
        name="sage_tc_root1",
    )(x, W_root1, b1.reshape(1, _D))


def _tc_layer1(r1, p, degp_t, W_neigh1):
    return pl.pallas_call(
        _layer1_body, grid=(_N // _R,),
        in_specs=[_row_spec, _p_spec, _degp_spec, _w_spec],
        out_specs=_row_spec,
        out_shape=jax.ShapeDtypeStruct((_N, _D), jnp.float32),
        name="sage_tc1",
    )(r1, p, degp_t, W_neigh1)


def _tc_root2(h1, W_root2, b2, W_out, b_out):
    return pl.pallas_call(
        _root2_body, grid=(_N // _R,),
        in_specs=[_row_spec, _w_spec, _b_spec, _wo_spec, _bo_spec],
        out_specs=[_row_spec, _out_spec],
        out_shape=[jax.ShapeDtypeStruct((_N, _D), jnp.float32),
                   jax.ShapeDtypeStruct((_N, _C), jnp.float32)],
        name="sage_tc_root2",
    )(h1, W_root2, b2.reshape(1, _D), W_out, b_out.reshape(1, _C))


def _tc_layer2(r2, o1, p, degp_t, W_neigh2, W_out):
    return pl.pallas_call(
        _layer2_body, grid=(_N // _R,),
        in_specs=[_row_spec, _out_spec, _p_spec, _degp_spec, _w_spec,
                  _wo_spec],
        out_specs=_out_spec,
        out_shape=jax.ShapeDtypeStruct((_N, _C), jnp.float32),
        name="sage_tc2",
    )(r2, o1, p, degp_t, W_neigh2, W_out)


@jax.jit
def kernel(x, edge_index, W_root1, W_neigh1, b1, W_root2, W_neigh2, b2,
           W_out, b_out):
    edge_flat = edge_index.reshape(2 * _E)
    p1, degp = _agg_deg(x, edge_flat)
    r1 = _tc_root1(x, W_root1, b1)
    degp_t = degp[:, :_N].T
    h1 = _tc_layer1(r1, p1, degp_t, W_neigh1)
    (p2,) = _agg(h1, edge_flat)
    r2, o1 = _tc_root2(h1, W_root2, b2, W_out, b_out)
    return _tc_layer2(r2, o1, p2, degp_t, W_neigh2, W_out)
